# Initial kernel scaffold; baseline (speedup 1.0000x reference)
import jax
import jax.numpy as jnp
from jax.experimental import pallas as pl


def kernel(h, edge_index, W, b, att_W, att_b):
    raise NotImplementedError("write your pallas kernel here")


# speedup vs baseline: 6.9967x; 6.9967x over previous
import functools

import jax
import jax.numpy as jnp
from jax import lax
from jax.experimental import pallas as pl
from jax.experimental.pallas import tpu as pltpu
from jax.experimental.pallas import tpu_sc as plsc


def _s_body(vt_ref, h_ref, c_ref, s_ref, lc_ref):
    s = lax.dot_general(
        vt_ref[...], h_ref[...], (((1,), (1,)), ((), ())),
        preferred_element_type=jnp.float32) + c_ref[...]
    s_ref[...] = s
    m = jnp.sum(jnp.max(s, axis=1))
    lc = jnp.maximum(m, 0.01 * m)
    lc_ref[...] = jnp.broadcast_to(lc, (128,))


def _out_body(p_ref, z_ref, o_ref):
    zsum = jnp.sum(z_ref[...])
    o_ref[...] = jnp.maximum(p_ref[0] + p_ref[1], 0.0) * (1.0 / zsum)


def _make_edge_kernel(n, e, d, n_cores, n_sub):
    nw = n_cores * n_sub
    ew = e // nw
    ch = 80
    nch = ew // ch
    groups = ch // 16
    nq = (n // 8) // n_sub
    rem = (n // 8) - nq * n_sub
    zq = 26
    nzc = nq // zq

    mesh = plsc.VectorSubcoreMesh(core_axis_name="c", subcore_axis_name="s")

    @functools.partial(
        pl.kernel,
        mesh=mesh,
        compiler_params=pltpu.CompilerParams(needs_layout_passes=False),
        out_type=[
            jax.ShapeDtypeStruct((n_cores, n, d), jnp.float32),
            jax.ShapeDtypeStruct((nw * 16,), jnp.float32),
        ],
        scratch_types=[
            pltpu.VMEM((n,), jnp.float32),
            pltpu.VMEM((n,), jnp.float32),
            pltpu.VMEM_SHARED((n, d), jnp.float32),
            pltpu.VMEM((ch, d), jnp.float32),
            pltpu.VMEM((ch,), jnp.int32),
            pltpu.VMEM((ch,), jnp.int32),
            pltpu.VMEM((ch,), jnp.float32),
            pltpu.VMEM((8, d), jnp.float32),
            pltpu.VMEM((16,), jnp.float32),
            pltpu.VMEM((128,), jnp.float32),
            pltpu.SemaphoreType.DMA,
        ],
    )
    def edge_kernel(h_hbm, s1_hbm, s2_hbm, src_hbm, dst_hbm, lc_hbm,
                    p_hbm, z_hbm,
                    s1_v, s2_v, acc, rows_v, si_v, di_v, u_v, zero_v, zv,
                    lc_v, sem):
        cid = lax.axis_index("c")
        sid = lax.axis_index("s")
        wid = cid * n_sub + sid

        pltpu.sync_copy(s1_hbm, s1_v)
        pltpu.sync_copy(s2_hbm, s2_v)

        def zfill_loop(i, _):
            for k in range(d // 16):
                zero_v[i, pl.ds(k * 16, 16)] = jnp.zeros((16,), jnp.float32)
            return 0
        lax.fori_loop(0, 8, zfill_loop, 0)
        r0 = pl.multiple_of(8 * (sid * nq + jnp.minimum(sid, rem)), 8)

        def zero_loop(i, _):
            pltpu.sync_copy(zero_v, acc.at[pl.ds(r0 + i * 8, 8)])
            return 0
        lax.fori_loop(0, nq, zero_loop, 0)

        @pl.when(sid < rem)
        def _():
            rr = pl.multiple_of(8 * (n_sub * nq + sid), 8)
            pltpu.sync_copy(zero_v, acc.at[pl.ds(rr, 8)])

        pltpu.sync_copy(lc_hbm, lc_v)
        lc = lc_v[pl.ds(0, 16)]

        plsc.subcore_barrier()

        e0 = wid * ew

        def chunk_loop(ci, zacc):
            base = e0 + ci * ch
            pltpu.sync_copy(src_hbm.at[pl.ds(base, ch)], si_v)
            pltpu.sync_copy(dst_hbm.at[pl.ds(base, ch)], di_v)
            pltpu.async_copy(h_hbm.at[si_v], rows_v, sem).wait()
            for g in range(groups):
                sl = pl.ds(g * 16, 16)
                z = (plsc.load_gather(s1_v, [si_v[sl]])
                     + plsc.load_gather(s2_v, [di_v[sl]]))
                lr = jnp.maximum(z, 0.01 * z)
                u = jnp.exp(lr - lc)
                zacc = zacc + u
                u_v[sl] = u
            def scale_loop(ei, _):
                w = plsc.load_gather(u_v, [jnp.zeros((16,), jnp.int32) + ei])
                for k in range(d // 16):
                    sl = pl.ds(k * 16, 16)
                    rows_v[ei, sl] = rows_v[ei, sl] * w
                return 0
            lax.fori_loop(0, ch, scale_loop, 0)
            pltpu.sync_copy(rows_v, acc.at[di_v], add=True)
            return zacc

        zsum = lax.fori_loop(0, nch, chunk_loop, jnp.zeros((16,), jnp.float32))
        zv[...] = zsum
        pltpu.sync_copy(zv, z_hbm.at[pl.ds(wid * 16, 16)])

        plsc.subcore_barrier()
        for j in range(nzc):
            sl = pl.ds(r0 + j * zq * 8, zq * 8)
            pltpu.sync_copy(acc.at[sl], p_hbm.at[cid, sl])

        @pl.when(sid < rem)
        def _():
            rr = pl.multiple_of(8 * (n_sub * nq + sid), 8)
            sl = pl.ds(rr, 8)
            pltpu.sync_copy(acc.at[sl], p_hbm.at[cid, sl])

    return edge_kernel


def kernel(h, edge_index, W, b, att_W, att_b):
    n, d_in = h.shape
    d_out = W.shape[0]
    e = edge_index.shape[1]

    a1 = att_W[0, :d_out]
    a2 = att_W[0, d_out:]
    vt = jnp.stack([W.T @ a1, W.T @ a2])
    c = jnp.stack([b @ a1 + att_b[0], b @ a2])[:, None]

    st, lc = pl.pallas_call(
        _s_body,
        out_shape=[jax.ShapeDtypeStruct((2, n), jnp.float32),
                   jax.ShapeDtypeStruct((128,), jnp.float32)],
    )(vt, h, c)

    edge_kernel = _make_edge_kernel(n, e, d_in, 2, 16)
    p, z = edge_kernel(h, st[0], st[1], edge_index[0], edge_index[1], lc)
    z = z.reshape(4, 128)

    out = pl.pallas_call(
        _out_body,
        out_shape=jax.ShapeDtypeStruct((n, d_in), jnp.float32),
    )(p, z)
    return out


# speedup vs baseline: 9.6154x; 1.3743x over previous
import functools

import jax
import jax.numpy as jnp
from jax import lax
from jax.experimental import pallas as pl
from jax.experimental.pallas import tpu as pltpu
from jax.experimental.pallas import tpu_sc as plsc


def _s_body(vt_ref, h_ref, c_ref, s_ref, lc_ref):
    s = lax.dot_general(
        vt_ref[...], h_ref[...], (((1,), (1,)), ((), ())),
        preferred_element_type=jnp.float32) + c_ref[...]
    s_ref[...] = s
    m = jnp.sum(jnp.max(s, axis=1))
    lc = jnp.maximum(m, 0.01 * m)
    lc_ref[...] = jnp.broadcast_to(lc, (128,))


def _out_body(p_ref, z_ref, o_ref):
    zsum = jnp.sum(z_ref[...])
    o_ref[...] = jnp.maximum(p_ref[0] + p_ref[1], 0.0) * (1.0 / zsum)


def _make_edge_kernel(n, e, d, n_cores, n_sub):
    nw = n_cores * n_sub
    ew = e // nw
    ch = 80
    nch = ew // ch
    groups = ch // 16
    nq = (n // 8) // n_sub
    rem = (n // 8) - nq * n_sub
    zq = 26
    nzc = nq // zq

    mesh = plsc.VectorSubcoreMesh(core_axis_name="c", subcore_axis_name="s")

    @functools.partial(
        pl.kernel,
        mesh=mesh,
        compiler_params=pltpu.CompilerParams(needs_layout_passes=False),
        out_type=[
            jax.ShapeDtypeStruct((n_cores, n, d), jnp.float32),
            jax.ShapeDtypeStruct((nw * 16,), jnp.float32),
        ],
        scratch_types=[
            pltpu.VMEM((n,), jnp.float32),
            pltpu.VMEM((n,), jnp.float32),
            pltpu.VMEM_SHARED((n, d), jnp.float32),
            pltpu.VMEM((ch, d), jnp.float32),
            pltpu.VMEM((ch, d), jnp.float32),
            pltpu.VMEM((ch,), jnp.int32),
            pltpu.VMEM((ch,), jnp.int32),
            pltpu.VMEM((ch,), jnp.int32),
            pltpu.VMEM((ch,), jnp.int32),
            pltpu.VMEM((ch,), jnp.int32),
            pltpu.VMEM((ch,), jnp.int32),
            pltpu.VMEM((ch,), jnp.float32),
            pltpu.VMEM((8, d), jnp.float32),
            pltpu.VMEM((16,), jnp.float32),
            pltpu.VMEM((128,), jnp.float32),
        ],
    )
    def edge_kernel(h_hbm, s1_hbm, s2_hbm, src_hbm, dst_hbm, lc_hbm,
                    p_hbm, z_hbm,
                    s1_v, s2_v, acc, rows0, rows1, si0, si1, di0, di1,
                    ds0, ds1, u_v, zero_v, zv, lc_v):
        cid = lax.axis_index("c")
        sid = lax.axis_index("s")
        wid = cid * n_sub + sid

        pltpu.sync_copy(s1_hbm, s1_v)
        pltpu.sync_copy(s2_hbm, s2_v)

        def zfill_loop(i, _):
            for k in range(d // 16):
                zero_v[i, pl.ds(k * 16, 16)] = jnp.zeros((16,), jnp.float32)
            return 0
        lax.fori_loop(0, 8, zfill_loop, 0)
        r0 = pl.multiple_of(8 * (sid * nq + jnp.minimum(sid, rem)), 8)

        def zero_loop(i, _):
            pltpu.sync_copy(zero_v, acc.at[pl.ds(r0 + i * 8, 8)])
            return 0
        lax.fori_loop(0, nq, zero_loop, 0)

        @pl.when(sid < rem)
        def _():
            rr = pl.multiple_of(8 * (n_sub * nq + sid), 8)
            pltpu.sync_copy(zero_v, acc.at[pl.ds(rr, 8)])

        pltpu.sync_copy(lc_hbm, lc_v)
        lc = lc_v[pl.ds(0, 16)]

        plsc.subcore_barrier()

        e0 = wid * ew

        def compute_u(si, di, dsc, zacc):
            for g in range(groups):
                sl = pl.ds(g * 16, 16)
                z = (plsc.load_gather(s1_v, [si[sl]])
                     + plsc.load_gather(s2_v, [di[sl]]))
                lr = jnp.maximum(z, 0.01 * z)
                u = jnp.exp(lr - lc)
                zacc = zacc + u
                u_v[sl] = u
                dsc[sl] = di[sl]
            return zacc

        def scale(rows):
            def _body(ei, _):
                w = plsc.load_gather(u_v, [jnp.zeros((16,), jnp.int32) + ei])
                for k in range(d // 16):
                    sl = pl.ds(k * 16, 16)
                    rows[ei, sl] = rows[ei, sl] * w
                return 0
            lax.fori_loop(0, ch, _body, 0)

        def idx_sync(ci, si, di):
            base = e0 + ci * ch
            pltpu.sync_copy(src_hbm.at[pl.ds(base, ch)], si)
            pltpu.sync_copy(dst_hbm.at[pl.ds(base, ch)], di)

        def process(ci, zacc, si_c, di_c, ds_c, rows_c, refill):
            zacc = compute_u(si_c, di_c, ds_c, zacc)
            scale(rows_c)
            pltpu.sync_copy(rows_c, acc.at[ds_c], add=True)
            if refill:
                idx_sync(ci + 2, si_c, di_c)
            return zacc

        def slot(zacc, ci, si_c, di_c, ds_c, rows_c, si_n, rows_n, refill):
            def scoped(gsem):
                pltpu.async_copy(h_hbm.at[si_n], rows_n, gsem)
                out = process(ci, zacc, si_c, di_c, ds_c, rows_c, refill)
                pltpu.make_async_copy(h_hbm.at[si_n], rows_n, gsem).wait()
                return out
            return pl.run_scoped(scoped, gsem=pltpu.SemaphoreType.DMA)

        idx_sync(0, si0, di0)
        pltpu.sync_copy(h_hbm.at[si0], rows0)
        idx_sync(1, si1, di1)
        zsum = process(0, jnp.zeros((16,), jnp.float32),
                       si0, di0, ds0, rows0, True)

        def pair_body(k, zacc):
            ci = 2 * k + 1
            zacc = slot(zacc, ci, si1, di1, ds1, rows1, si0, rows0, True)
            return slot(zacc, ci + 1, si0, di0, ds0, rows0, si1, rows1, True)

        zsum = lax.fori_loop(0, (nch - 3) // 2, pair_body, zsum)
        zsum = slot(zsum, nch - 2, si1, di1, ds1, rows1, si0, rows0, False)
        zsum = process(nch - 1, zsum, si0, di0, ds0, rows0, False)
        zv[...] = zsum
        pltpu.sync_copy(zv, z_hbm.at[pl.ds(wid * 16, 16)])

        plsc.subcore_barrier()
        for j in range(nzc):
            sl = pl.ds(r0 + j * zq * 8, zq * 8)
            pltpu.sync_copy(acc.at[sl], p_hbm.at[cid, sl])

        @pl.when(sid < rem)
        def _():
            rr = pl.multiple_of(8 * (n_sub * nq + sid), 8)
            sl = pl.ds(rr, 8)
            pltpu.sync_copy(acc.at[sl], p_hbm.at[cid, sl])

    return edge_kernel


def kernel(h, edge_index, W, b, att_W, att_b):
    n, d_in = h.shape
    d_out = W.shape[0]
    e = edge_index.shape[1]

    a1 = att_W[0, :d_out]
    a2 = att_W[0, d_out:]
    vt = jnp.stack([W.T @ a1, W.T @ a2])
    c = jnp.stack([b @ a1 + att_b[0], b @ a2])[:, None]

    st, lc = pl.pallas_call(
        _s_body,
        out_shape=[jax.ShapeDtypeStruct((2, n), jnp.float32),
                   jax.ShapeDtypeStruct((128,), jnp.float32)],
    )(vt, h, c)

    edge_kernel = _make_edge_kernel(n, e, d_in, 2, 16)
    p, z = edge_kernel(h, st[0], st[1], edge_index[0], edge_index[1], lc)
    z = z.reshape(4, 128)

    out = pl.pallas_call(
        _out_body,
        out_shape=jax.ShapeDtypeStruct((n, d_in), jnp.float32),
    )(p, z)
    return out
